# baseline (speedup 1.0000x reference)
import jax
import jax.numpy as jnp
from jax import lax
from jax.experimental import pallas as pl
from jax.experimental.pallas import tpu as pltpu
from jax.experimental.pallas import tpu_sc as plsc

N = 10000
N2 = 10240
E = 160000
E2 = 163840
DIN = 256
DHID = 512
DOUT = 256
HALF = 128
NT = 16
_f32 = jnp.float32

CH = 128
NB = 8
NCHUNK = E2 // (NT * CH)
NBLK = NCHUNK // NB
RPT = N2 // NT


def _sc_agg_body(table, srcs, dsts, agg_out, accum, sidx, didx,
                 gbuf0, gbuf1, gsem0, gsem1, isem):
    c = lax.axis_index("c")
    s = lax.axis_index("s")
    base = s * RPT

    def _zrow(i, _):
        for k in range(HALF // 16):
            gbuf0[i, pl.ds(k * 16, 16)] = jnp.zeros((16,), _f32)
        return 0
    lax.fori_loop(0, CH, _zrow, 0)
    for q in range(RPT // CH):
        pltpu.sync_copy(gbuf0, accum.at[pl.ds(base + q * CH, CH)])

    plsc.subcore_barrier()

    def _idx_start(b, par):
        pltpu.async_copy(srcs.at[s, pl.ds(b * NB, NB)], sidx.at[par], isem)
        pltpu.async_copy(dsts.at[s, pl.ds(b * NB, NB)], didx.at[par], isem)

    def _idx_wait(b, par):
        pltpu.make_async_copy(srcs.at[s, pl.ds(b * NB, NB)], sidx.at[par],
                              isem).wait()
        pltpu.make_async_copy(dsts.at[s, pl.ds(b * NB, NB)], didx.at[par],
                              isem).wait()

    _idx_start(0, 0)

    def _blk(b, _):
        par = lax.rem(b, 2)
        _idx_wait(b, par)

        @pl.when(b + 1 < NBLK)
        def _():
            _idx_start(b + 1, 1 - par)

        si = sidx.at[par]
        di = didx.at[par]

        def _fix(j, _):
            for k in range(CH // 16):
                sl = si[j, pl.ds(k * 16, 16)]
                si[j, pl.ds(k * 16, 16)] = sl * 2 + c
            return 0
        lax.fori_loop(0, NB, _fix, 0)

        bufs = (gbuf0, gbuf1)
        sems = (gsem0, gsem1)
        pltpu.async_copy(table.at[si.at[0]], bufs[0], sems[0])
        for j in range(NB):
            bj, sj = bufs[j % 2], sems[j % 2]
            pltpu.make_async_copy(table.at[si.at[j]], bj, sj).wait()
            if j + 1 < NB:
                pltpu.async_copy(table.at[si.at[j + 1]],
                                 bufs[(j + 1) % 2], sems[(j + 1) % 2])
            pltpu.sync_copy(bj, accum.at[di.at[j]], add=True)
        return 0
    lax.fori_loop(0, NBLK, _blk, 0)

    plsc.subcore_barrier()

    pltpu.sync_copy(accum.at[pl.ds(base, RPT)],
                    agg_out.at[pl.ds(base, RPT), c])


def _sc_deg_body(dsts, deg_out, deg2d, didx, ones_v, dsem):
    c = lax.axis_index("c")
    s = lax.axis_index("s")
    base = s * RPT

    def _zrow(i, _):
        for k in range(HALF // 16):
            ones_v[i, pl.ds(k * 16, 16)] = jnp.zeros((16,), _f32)
        return 0
    lax.fori_loop(0, CH, _zrow, 0)
    for q in range(RPT // CH):
        pltpu.sync_copy(ones_v, deg2d.at[pl.ds(base + q * CH, CH)])

    def _orow(i, _):
        for k in range(HALF // 16):
            ones_v[i, pl.ds(k * 16, 16)] = jnp.full((16,), 1.0, _f32)
        return 0
    lax.fori_loop(0, CH, _orow, 0)

    plsc.subcore_barrier()

    def _blk(b, _):
        pltpu.sync_copy(dsts.at[s, pl.ds(b * NB, NB)], didx)
        for j in range(NB):
            pltpu.async_copy(ones_v, deg2d.at[didx.at[j]], dsem, add=True)
        for j in range(NB):
            pltpu.make_async_copy(ones_v, deg2d.at[didx.at[j]], dsem).wait()
        return 0
    lax.fori_loop(0, NBLK, _blk, 0)

    plsc.subcore_barrier()

    @pl.when(c == 0)
    def _():
        pltpu.sync_copy(deg2d.at[pl.ds(base, RPT)],
                        deg_out.at[pl.ds(base, RPT)])


_SC_MESH = plsc.VectorSubcoreMesh(core_axis_name="c", subcore_axis_name="s")

_sc_agg = pl.kernel(
    _sc_agg_body,
    out_type=[jax.ShapeDtypeStruct((N2, 2, HALF), _f32)],
    mesh=_SC_MESH,
    scratch_types=[
        pltpu.VMEM_SHARED((N2, HALF), _f32),
        pltpu.VMEM((2, NB, CH), jnp.int32),
        pltpu.VMEM((2, NB, CH), jnp.int32),
        pltpu.VMEM((CH, HALF), _f32),
        pltpu.VMEM((CH, HALF), _f32),
        pltpu.SemaphoreType.DMA,
        pltpu.SemaphoreType.DMA,
        pltpu.SemaphoreType.DMA,
    ],
)

_sc_deg = pl.kernel(
    _sc_deg_body,
    out_type=[jax.ShapeDtypeStruct((N2, HALF), _f32)],
    mesh=_SC_MESH,
    scratch_types=[
        pltpu.VMEM_SHARED((N2, HALF), _f32),
        pltpu.VMEM((NB, CH), jnp.int32),
        pltpu.VMEM((CH, HALF), _f32),
        pltpu.SemaphoreType.DMA,
    ],
)


BR1 = 512
BR3 = 400


def _tc1_body(a, deg, x, wl, wr, b, out):
    inv = 1.0 / jnp.maximum(deg[:, 0:1], 1.0)
    wlb = wl[...].astype(jnp.bfloat16)
    acc = jnp.dot((a[:, 0, :] * inv).astype(jnp.bfloat16), wlb[:HALF],
                  preferred_element_type=_f32)
    acc += jnp.dot((a[:, 1, :] * inv).astype(jnp.bfloat16), wlb[HALF:],
                   preferred_element_type=_f32)
    acc += jnp.dot(x[...].astype(jnp.bfloat16),
                   wr[...].astype(jnp.bfloat16), preferred_element_type=_f32)
    out[...] = jnp.maximum(acc + b[...], 0.0).astype(jnp.bfloat16)


def _tc1(agg, deg, x_p, w1l, w1r, b1):
    return pl.pallas_call(
        _tc1_body,
        grid=(N2 // BR1,),
        in_specs=[
            pl.BlockSpec((BR1, 2, HALF), lambda i: (i, 0, 0)),
            pl.BlockSpec((BR1, HALF), lambda i: (i, 0)),
            pl.BlockSpec((BR1, DIN), lambda i: (i, 0)),
            pl.BlockSpec((DIN, DHID), lambda i: (0, 0)),
            pl.BlockSpec((DIN, DHID), lambda i: (0, 0)),
            pl.BlockSpec((1, DHID), lambda i: (0, 0)),
        ],
        out_specs=pl.BlockSpec((BR1, DHID), lambda i: (i, 0)),
        out_shape=jax.ShapeDtypeStruct((N2, DHID), jnp.bfloat16),
    )(agg, deg, x_p, w1l, w1r, b1)


def _tc2_body(h, wl, wr, b, p_out, r_out):
    hh = h[...]
    p = jnp.dot(hh, wl[...].astype(jnp.bfloat16), preferred_element_type=_f32)
    r = jnp.dot(hh, wr[...].astype(jnp.bfloat16),
                preferred_element_type=_f32) + b[...]
    p_out[:, 0, :] = p[:, :HALF]
    p_out[:, 1, :] = p[:, HALF:]
    r_out[:, 0, :] = r[:, :HALF].astype(jnp.bfloat16)
    r_out[:, 1, :] = r[:, HALF:].astype(jnp.bfloat16)


def _tc2(h, w2l, w2r, b2):
    return pl.pallas_call(
        _tc2_body,
        grid=(N2 // BR1,),
        in_specs=[
            pl.BlockSpec((BR1, DHID), lambda i: (i, 0)),
            pl.BlockSpec((DHID, DOUT), lambda i: (0, 0)),
            pl.BlockSpec((DHID, DOUT), lambda i: (0, 0)),
            pl.BlockSpec((1, DOUT), lambda i: (0, 0)),
        ],
        out_specs=[
            pl.BlockSpec((BR1, 2, HALF), lambda i: (i, 0, 0)),
            pl.BlockSpec((BR1, 2, HALF), lambda i: (i, 0, 0)),
        ],
        out_shape=[
            jax.ShapeDtypeStruct((N2, 2, HALF), _f32),
            jax.ShapeDtypeStruct((N2, 2, HALF), jnp.bfloat16),
        ],
    )(h, w2l, w2r, b2)


def _tc3_body(a, deg, r, z):
    inv = 1.0 / jnp.maximum(deg[:, 0:1], 1.0)
    rr = r[...].astype(_f32)
    z[...] = jnp.concatenate(
        [a[:, 0, :] * inv + rr[:, 0, :], a[:, 1, :] * inv + rr[:, 1, :]],
        axis=1)


def _tc3(agg2, deg, r3):
    return pl.pallas_call(
        _tc3_body,
        grid=(N // BR3,),
        in_specs=[
            pl.BlockSpec((BR3, 2, HALF), lambda i: (i, 0, 0)),
            pl.BlockSpec((BR3, HALF), lambda i: (i, 0)),
            pl.BlockSpec((BR3, 2, HALF), lambda i: (i, 0, 0)),
        ],
        out_specs=pl.BlockSpec((BR3, DOUT), lambda i: (i, 0)),
        out_shape=jax.ShapeDtypeStruct((N, DOUT), _f32),
    )(agg2, deg, r3)


def kernel(x, edge_index, W1_l, W1_r, b1, W2_l, W2_r, b2):
    src = edge_index[0].astype(jnp.int32)
    dst = edge_index[1].astype(jnp.int32)
    padi = (jnp.arange(E2 - E, dtype=jnp.int32) % 16) + N
    srcs = jnp.concatenate([src, padi]).reshape(NT, NCHUNK, CH)
    dsts = jnp.concatenate([dst, padi]).reshape(NT, NCHUNK, CH)

    x_p = jnp.pad(x, ((0, N2 - N), (0, 0)))
    xflat = x_p.reshape(2 * N2, HALF)

    (deg,) = _sc_deg(dsts)
    (agg1,) = _sc_agg(xflat, srcs, dsts)
    h = _tc1(agg1, deg, x_p, W1_l, W1_r, b1.reshape(1, DHID))
    p3, r3 = _tc2(h, W2_l, W2_r, b2.reshape(1, DOUT))
    (agg2,) = _sc_agg(p3.reshape(2 * N2, HALF), srcs, dsts)
    return _tc3(agg2, deg, r3)


# speedup vs baseline: 1.0685x; 1.0685x over previous
import jax
import jax.numpy as jnp
from jax import lax
from jax.experimental import pallas as pl
from jax.experimental.pallas import tpu as pltpu
from jax.experimental.pallas import tpu_sc as plsc

N = 10000
N2 = 10240
E = 160000
E2 = 163840
DIN = 256
DHID = 512
DOUT = 256
HALF = 128
NT = 16
_f32 = jnp.float32

CH = 128
NB = 8
NCHUNK = E2 // (NT * CH)
NBLK = NCHUNK // NB
RPT = N2 // NT


def _sc_agg_body(table, srcs, dsts, agg_out, accum, sidx, didx,
                 gbuf0, gbuf1, gsem0, gsem1, isem):
    c = lax.axis_index("c")
    s = lax.axis_index("s")
    base = s * RPT

    def _zrow(i, _):
        for k in range(HALF // 16):
            gbuf0[i, pl.ds(k * 16, 16)] = jnp.zeros((16,), _f32)
        return 0
    lax.fori_loop(0, CH, _zrow, 0)
    for q in range(RPT // CH):
        pltpu.sync_copy(gbuf0, accum.at[pl.ds(base + q * CH, CH)])

    plsc.subcore_barrier()

    def _idx_start(b, par):
        pltpu.async_copy(srcs.at[s, pl.ds(b * NB, NB)], sidx.at[par], isem)
        pltpu.async_copy(dsts.at[s, pl.ds(b * NB, NB)], didx.at[par], isem)

    def _idx_wait(b, par):
        pltpu.make_async_copy(srcs.at[s, pl.ds(b * NB, NB)], sidx.at[par],
                              isem).wait()
        pltpu.make_async_copy(dsts.at[s, pl.ds(b * NB, NB)], didx.at[par],
                              isem).wait()

    _idx_start(0, 0)

    def _blk(b, _):
        par = lax.rem(b, 2)
        _idx_wait(b, par)

        @pl.when(b + 1 < NBLK)
        def _():
            _idx_start(b + 1, 1 - par)

        si = sidx.at[par]
        di = didx.at[par]

        def _fix(j, _):
            for k in range(CH // 16):
                sl = si[j, pl.ds(k * 16, 16)]
                si[j, pl.ds(k * 16, 16)] = sl * 2 + c
            return 0
        lax.fori_loop(0, NB, _fix, 0)

        bufs = (gbuf0, gbuf1)
        sems = (gsem0, gsem1)
        pltpu.async_copy(table.at[si.at[0]], bufs[0], sems[0])
        for j in range(NB):
            bj, sj = bufs[j % 2], sems[j % 2]
            pltpu.make_async_copy(table.at[si.at[j]], bj, sj).wait()
            if j + 1 < NB:
                pltpu.async_copy(table.at[si.at[j + 1]],
                                 bufs[(j + 1) % 2], sems[(j + 1) % 2])
            pltpu.sync_copy(bj, accum.at[di.at[j]], add=True)
        return 0
    lax.fori_loop(0, NBLK, _blk, 0)

    plsc.subcore_barrier()

    pltpu.sync_copy(accum.at[pl.ds(base, RPT)],
                    agg_out.at[pl.ds(base, RPT), c])


def _sc_deg_body(dsts, deg_out, deg2d, didx, ones_v, dsem):
    c = lax.axis_index("c")
    s = lax.axis_index("s")
    base = s * RPT

    def _zrow(i, _):
        for k in range(HALF // 16):
            ones_v[i, pl.ds(k * 16, 16)] = jnp.zeros((16,), _f32)
        return 0
    lax.fori_loop(0, CH, _zrow, 0)
    for q in range(RPT // CH):
        pltpu.sync_copy(ones_v, deg2d.at[pl.ds(base + q * CH, CH)])

    def _orow(i, _):
        for k in range(HALF // 16):
            ones_v[i, pl.ds(k * 16, 16)] = jnp.full((16,), 1.0, _f32)
        return 0
    lax.fori_loop(0, CH, _orow, 0)

    plsc.subcore_barrier()

    def _blk(b, _):
        blk = c * (NBLK // 2) + b
        pltpu.sync_copy(dsts.at[s, pl.ds(blk * NB, NB)], didx)
        for j in range(NB):
            pltpu.async_copy(ones_v, deg2d.at[didx.at[j]], dsem, add=True)
        for j in range(NB):
            pltpu.make_async_copy(ones_v, deg2d.at[didx.at[j]], dsem).wait()
        return 0
    lax.fori_loop(0, NBLK // 2, _blk, 0)

    plsc.subcore_barrier()

    pltpu.sync_copy(deg2d.at[pl.ds(base, RPT)],
                    deg_out.at[pl.ds(base, RPT), c])


_SC_MESH = plsc.VectorSubcoreMesh(core_axis_name="c", subcore_axis_name="s")

_sc_agg = pl.kernel(
    _sc_agg_body,
    out_type=[jax.ShapeDtypeStruct((N2, 2, HALF), _f32)],
    mesh=_SC_MESH,
    scratch_types=[
        pltpu.VMEM_SHARED((N2, HALF), _f32),
        pltpu.VMEM((2, NB, CH), jnp.int32),
        pltpu.VMEM((2, NB, CH), jnp.int32),
        pltpu.VMEM((CH, HALF), _f32),
        pltpu.VMEM((CH, HALF), _f32),
        pltpu.SemaphoreType.DMA,
        pltpu.SemaphoreType.DMA,
        pltpu.SemaphoreType.DMA,
    ],
)

_sc_deg = pl.kernel(
    _sc_deg_body,
    out_type=[jax.ShapeDtypeStruct((N2, 2, HALF), _f32)],
    mesh=_SC_MESH,
    scratch_types=[
        pltpu.VMEM_SHARED((N2, HALF), _f32),
        pltpu.VMEM((NB, CH), jnp.int32),
        pltpu.VMEM((CH, HALF), _f32),
        pltpu.SemaphoreType.DMA,
    ],
)


BR = 400
_bf16 = jnp.bfloat16


def _inv_deg(deg):
    return 1.0 / jnp.maximum(deg[:, 0, 0:1] + deg[:, 1, 0:1], 1.0)


def _tc1_body(a, deg, x, wl, wr, b, out):
    inv = _inv_deg(deg)
    acc = jnp.dot((a[:, 0, :] * inv).astype(_bf16), wl[:HALF],
                  preferred_element_type=_f32)
    acc += jnp.dot((a[:, 1, :] * inv).astype(_bf16), wl[HALF:],
                   preferred_element_type=_f32)
    acc += jnp.dot(x[...].astype(_bf16), wr[...], preferred_element_type=_f32)
    out[...] = jnp.maximum(acc + b[...], 0.0).astype(_bf16)


def _tc1(agg, deg, x, w1l, w1r, b1):
    return pl.pallas_call(
        _tc1_body,
        grid=(N // BR,),
        in_specs=[
            pl.BlockSpec((BR, 2, HALF), lambda i: (i, 0, 0)),
            pl.BlockSpec((BR, 2, HALF), lambda i: (i, 0, 0)),
            pl.BlockSpec((BR, DIN), lambda i: (i, 0)),
            pl.BlockSpec((DIN, DHID), lambda i: (0, 0)),
            pl.BlockSpec((DIN, DHID), lambda i: (0, 0)),
            pl.BlockSpec((1, DHID), lambda i: (0, 0)),
        ],
        out_specs=pl.BlockSpec((BR, DHID), lambda i: (i, 0)),
        out_shape=jax.ShapeDtypeStruct((N, DHID), _bf16),
    )(agg, deg, x, w1l, w1r, b1)


def _tc2p_body(h, wl, p_out):
    pp = jnp.dot(h[...], wl[...], preferred_element_type=_f32)
    p_out[:, 0, :] = pp[:, :HALF]
    p_out[:, 1, :] = pp[:, HALF:]


def _tc2p(h, w2l):
    return pl.pallas_call(
        _tc2p_body,
        grid=(N // BR,),
        in_specs=[
            pl.BlockSpec((BR, DHID), lambda i: (i, 0)),
            pl.BlockSpec((DHID, DOUT), lambda i: (0, 0)),
        ],
        out_specs=pl.BlockSpec((BR, 2, HALF), lambda i: (i, 0, 0)),
        out_shape=jax.ShapeDtypeStruct((N, 2, HALF), _f32),
    )(h, w2l)


def _tc2r_body(h, wr, b, r_out):
    rr = jnp.dot(h[...], wr[...], preferred_element_type=_f32) + b[...]
    r_out[:, 0, :] = rr[:, :HALF].astype(_bf16)
    r_out[:, 1, :] = rr[:, HALF:].astype(_bf16)


def _tc2r(h, w2r, b2):
    return pl.pallas_call(
        _tc2r_body,
        grid=(N // BR,),
        in_specs=[
            pl.BlockSpec((BR, DHID), lambda i: (i, 0)),
            pl.BlockSpec((DHID, DOUT), lambda i: (0, 0)),
            pl.BlockSpec((1, DOUT), lambda i: (0, 0)),
        ],
        out_specs=pl.BlockSpec((BR, 2, HALF), lambda i: (i, 0, 0)),
        out_shape=jax.ShapeDtypeStruct((N, 2, HALF), _bf16),
    )(h, w2r, b2)


def _tc3_body(a, deg, r, z):
    inv = _inv_deg(deg)
    rr = r[...].astype(_f32)
    z[...] = jnp.concatenate(
        [a[:, 0, :] * inv + rr[:, 0, :], a[:, 1, :] * inv + rr[:, 1, :]],
        axis=1)


def _tc3(agg2, deg, r3):
    return pl.pallas_call(
        _tc3_body,
        grid=(N // BR,),
        in_specs=[
            pl.BlockSpec((BR, 2, HALF), lambda i: (i, 0, 0)),
            pl.BlockSpec((BR, 2, HALF), lambda i: (i, 0, 0)),
            pl.BlockSpec((BR, 2, HALF), lambda i: (i, 0, 0)),
        ],
        out_specs=pl.BlockSpec((BR, DOUT), lambda i: (i, 0)),
        out_shape=jax.ShapeDtypeStruct((N, DOUT), _f32),
    )(agg2, deg, r3)


def kernel(x, edge_index, W1_l, W1_r, b1, W2_l, W2_r, b2):
    src = edge_index[0].astype(jnp.int32)
    dst = edge_index[1].astype(jnp.int32)
    pad = jnp.arange(E2 - E, dtype=jnp.int32) % 16
    srcs = jnp.concatenate([src, pad]).reshape(NT, NCHUNK, CH)
    dsts = jnp.concatenate([dst, pad + N]).reshape(NT, NCHUNK, CH)

    xflat = x.reshape(2 * N, HALF)
    w1l = W1_l.astype(_bf16)
    w1r = W1_r.astype(_bf16)

    (deg,) = _sc_deg(dsts)
    (agg1,) = _sc_agg(xflat, srcs, dsts)
    h = _tc1(agg1, deg, x, w1l, w1r, b1.reshape(1, DHID))
    p3 = _tc2p(h, W2_l.astype(_bf16))
    r3 = _tc2r(h, W2_r.astype(_bf16), b2.reshape(1, DOUT))
    (agg2,) = _sc_agg(p3.reshape(2 * N, HALF), srcs, dsts)
    return _tc3(agg2, deg, r3)


# speedup vs baseline: 1.1503x; 1.0765x over previous
import jax
import jax.numpy as jnp
from jax import lax
from jax.experimental import pallas as pl
from jax.experimental.pallas import tpu as pltpu
from jax.experimental.pallas import tpu_sc as plsc

N = 10000
N2 = 10240
E = 160000
E2 = 163840
DIN = 256
DHID = 512
DOUT = 256
HALF = 128
NT = 16
_f32 = jnp.float32

CH = 128
NB = 8
NCHUNK = E2 // (NT * CH)
NBLK = NCHUNK // NB
RPT = N2 // NT


def _sc_agg_body(table, srcs, dsts, dep, agg_out, accum, sidx, didx,
                 gbuf0, gbuf1, gsem0, gsem1, isem):
    del dep
    c = lax.axis_index("c")
    s = lax.axis_index("s")
    base = s * RPT

    def _zrow(i, _):
        for k in range(HALF // 16):
            gbuf0[i, pl.ds(k * 16, 16)] = jnp.zeros((16,), _f32)
        return 0
    lax.fori_loop(0, CH, _zrow, 0)
    for q in range(RPT // CH):
        pltpu.sync_copy(gbuf0, accum.at[pl.ds(base + q * CH, CH)])

    plsc.subcore_barrier()

    def _idx_start(b, par):
        pltpu.async_copy(srcs.at[s, pl.ds(b * NB, NB)], sidx.at[par], isem)
        pltpu.async_copy(dsts.at[s, pl.ds(b * NB, NB)], didx.at[par], isem)

    def _idx_wait(b, par):
        pltpu.make_async_copy(srcs.at[s, pl.ds(b * NB, NB)], sidx.at[par],
                              isem).wait()
        pltpu.make_async_copy(dsts.at[s, pl.ds(b * NB, NB)], didx.at[par],
                              isem).wait()

    _idx_start(0, 0)

    def _blk(b, _):
        par = lax.rem(b, 2)
        _idx_wait(b, par)

        @pl.when(b + 1 < NBLK)
        def _():
            _idx_start(b + 1, 1 - par)

        si = sidx.at[par]
        di = didx.at[par]

        def _fix(j, _):
            for k in range(CH // 16):
                sl = si[j, pl.ds(k * 16, 16)]
                si[j, pl.ds(k * 16, 16)] = sl * 2 + c
            return 0
        lax.fori_loop(0, NB, _fix, 0)

        bufs = (gbuf0, gbuf1)
        sems = (gsem0, gsem1)
        pltpu.async_copy(table.at[si.at[0]], bufs[0], sems[0])
        for j in range(NB):
            bj, sj = bufs[j % 2], sems[j % 2]
            pltpu.make_async_copy(table.at[si.at[j]], bj, sj).wait()
            if j + 1 < NB:
                pltpu.async_copy(table.at[si.at[j + 1]],
                                 bufs[(j + 1) % 2], sems[(j + 1) % 2])
            pltpu.sync_copy(bj, accum.at[di.at[j]], add=True)
        return 0
    lax.fori_loop(0, NBLK, _blk, 0)

    plsc.subcore_barrier()

    pltpu.sync_copy(accum.at[pl.ds(base, RPT)],
                    agg_out.at[pl.ds(base, RPT), c])


def _sc_deg_body(dsts, deg_out, deg2d, didx, ones_v, dsem):
    c = lax.axis_index("c")
    s = lax.axis_index("s")
    base = s * RPT

    def _zrow(i, _):
        for k in range(HALF // 16):
            ones_v[i, pl.ds(k * 16, 16)] = jnp.zeros((16,), _f32)
        return 0
    lax.fori_loop(0, CH, _zrow, 0)
    for q in range(RPT // CH):
        pltpu.sync_copy(ones_v, deg2d.at[pl.ds(base + q * CH, CH)])

    def _orow(i, _):
        for k in range(HALF // 16):
            ones_v[i, pl.ds(k * 16, 16)] = jnp.full((16,), 1.0, _f32)
        return 0
    lax.fori_loop(0, CH, _orow, 0)

    plsc.subcore_barrier()

    def _blk(b, _):
        blk = c * (NBLK // 2) + b
        pltpu.sync_copy(dsts.at[s, pl.ds(blk * NB, NB)], didx)
        for j in range(NB):
            pltpu.async_copy(ones_v, deg2d.at[didx.at[j]], dsem, add=True)
        for j in range(NB):
            pltpu.make_async_copy(ones_v, deg2d.at[didx.at[j]], dsem).wait()
        return 0
    lax.fori_loop(0, NBLK // 2, _blk, 0)

    plsc.subcore_barrier()

    pltpu.sync_copy(deg2d.at[pl.ds(base, RPT)],
                    deg_out.at[pl.ds(base, RPT), c])


_SC_MESH = plsc.VectorSubcoreMesh(core_axis_name="c", subcore_axis_name="s")

_sc_agg = pl.kernel(
    _sc_agg_body,
    out_type=[jax.ShapeDtypeStruct((N2, 2, HALF), _f32)],
    mesh=_SC_MESH,
    scratch_types=[
        pltpu.VMEM_SHARED((N2, HALF), _f32),
        pltpu.VMEM((2, NB, CH), jnp.int32),
        pltpu.VMEM((2, NB, CH), jnp.int32),
        pltpu.VMEM((CH, HALF), _f32),
        pltpu.VMEM((CH, HALF), _f32),
        pltpu.SemaphoreType.DMA,
        pltpu.SemaphoreType.DMA,
        pltpu.SemaphoreType.DMA,
    ],
)

_sc_deg = pl.kernel(
    _sc_deg_body,
    out_type=[jax.ShapeDtypeStruct((N2, 2, HALF), _f32)],
    mesh=_SC_MESH,
    scratch_types=[
        pltpu.VMEM_SHARED((N2, HALF), _f32),
        pltpu.VMEM((NB, CH), jnp.int32),
        pltpu.VMEM((CH, HALF), _f32),
        pltpu.SemaphoreType.DMA,
    ],
)


BR = 400
_bf16 = jnp.bfloat16


def _tcinv_body(deg, inv):
    inv[...] = 1.0 / jnp.maximum(deg[:, 0, 0:1] + deg[:, 1, 0:1], 1.0)


def _tcinv(deg):
    return pl.pallas_call(
        _tcinv_body,
        grid=(N // BR,),
        in_specs=[pl.BlockSpec((BR, 2, HALF), lambda i: (i, 0, 0))],
        out_specs=pl.BlockSpec((BR, 1), lambda i: (i, 0)),
        out_shape=jax.ShapeDtypeStruct((N, 1), _f32),
    )(deg)


def _tc1_body(a, inv, x, wl, wr, b, w2l, h_out, p_out):
    iv = inv[...]
    acc = jnp.dot((a[:, 0, :] * iv).astype(_bf16), wl[:HALF],
                  preferred_element_type=_f32)
    acc += jnp.dot((a[:, 1, :] * iv).astype(_bf16), wl[HALF:],
                   preferred_element_type=_f32)
    acc += jnp.dot(x[...].astype(_bf16), wr[...], preferred_element_type=_f32)
    hb = jnp.maximum(acc + b[...], 0.0).astype(_bf16)
    h_out[...] = hb
    pp = jnp.dot(hb, w2l[...], preferred_element_type=_f32)
    p_out[:, 0, :] = pp[:, :HALF]
    p_out[:, 1, :] = pp[:, HALF:]


def _tc1(agg, inv, x, w1l, w1r, b1, w2l):
    return pl.pallas_call(
        _tc1_body,
        grid=(N // BR,),
        in_specs=[
            pl.BlockSpec((BR, 2, HALF), lambda i: (i, 0, 0)),
            pl.BlockSpec((BR, 1), lambda i: (i, 0)),
            pl.BlockSpec((BR, DIN), lambda i: (i, 0)),
            pl.BlockSpec((DIN, DHID), lambda i: (0, 0)),
            pl.BlockSpec((DIN, DHID), lambda i: (0, 0)),
            pl.BlockSpec((1, DHID), lambda i: (0, 0)),
            pl.BlockSpec((DHID, DOUT), lambda i: (0, 0)),
        ],
        out_specs=[
            pl.BlockSpec((BR, DHID), lambda i: (i, 0)),
            pl.BlockSpec((BR, 2, HALF), lambda i: (i, 0, 0)),
        ],
        out_shape=[
            jax.ShapeDtypeStruct((N, DHID), _bf16),
            jax.ShapeDtypeStruct((N, 2, HALF), _f32),
        ],
    )(agg, inv, x, w1l, w1r, b1, w2l)


def _tc2r_body(h, wr, b, r_out):
    rr = jnp.dot(h[...], wr[...], preferred_element_type=_f32) + b[...]
    r_out[:, 0, :] = rr[:, :HALF].astype(_bf16)
    r_out[:, 1, :] = rr[:, HALF:].astype(_bf16)


def _tc2r(h, w2r, b2):
    return pl.pallas_call(
        _tc2r_body,
        grid=(N // BR,),
        in_specs=[
            pl.BlockSpec((BR, DHID), lambda i: (i, 0)),
            pl.BlockSpec((DHID, DOUT), lambda i: (0, 0)),
            pl.BlockSpec((1, DOUT), lambda i: (0, 0)),
        ],
        out_specs=pl.BlockSpec((BR, 2, HALF), lambda i: (i, 0, 0)),
        out_shape=jax.ShapeDtypeStruct((N, 2, HALF), _bf16),
    )(h, w2r, b2)


def _tc3_body(a, inv, r, z):
    iv = inv[...]
    rr = r[...].astype(_f32)
    z[...] = jnp.concatenate(
        [a[:, 0, :] * iv + rr[:, 0, :], a[:, 1, :] * iv + rr[:, 1, :]],
        axis=1)


def _tc3(agg2, inv, r3):
    return pl.pallas_call(
        _tc3_body,
        grid=(N // BR,),
        in_specs=[
            pl.BlockSpec((BR, 2, HALF), lambda i: (i, 0, 0)),
            pl.BlockSpec((BR, 1), lambda i: (i, 0)),
            pl.BlockSpec((BR, 2, HALF), lambda i: (i, 0, 0)),
        ],
        out_specs=pl.BlockSpec((BR, DOUT), lambda i: (i, 0)),
        out_shape=jax.ShapeDtypeStruct((N, DOUT), _f32),
    )(agg2, inv, r3)


def kernel(x, edge_index, W1_l, W1_r, b1, W2_l, W2_r, b2):
    src = edge_index[0].astype(jnp.int32)
    dst = edge_index[1].astype(jnp.int32)
    pad = jnp.arange(E2 - E, dtype=jnp.int32) % 16
    srcs = jnp.concatenate([src, pad]).reshape(NT, NCHUNK, CH)
    dsts = jnp.concatenate([dst, pad + N]).reshape(NT, NCHUNK, CH)

    xflat = x.reshape(2 * N, HALF)

    (deg,) = _sc_deg(dsts)
    inv = _tcinv(deg)
    (agg1,) = _sc_agg(xflat, srcs, dsts, deg)
    h, p3 = _tc1(agg1, inv, x, W1_l.astype(_bf16), W1_r.astype(_bf16),
                 b1.reshape(1, DHID), W2_l.astype(_bf16))
    r3 = _tc2r(h, W2_r.astype(_bf16), b2.reshape(1, DOUT))
    (agg2,) = _sc_agg(p3.reshape(2 * N, HALF), srcs, dsts, deg)
    return _tc3(agg2, inv, r3)


# speedup vs baseline: 1.2202x; 1.0608x over previous
import jax
import jax.numpy as jnp
from jax import lax
from jax.experimental import pallas as pl
from jax.experimental.pallas import tpu as pltpu
from jax.experimental.pallas import tpu_sc as plsc

N = 10000
N2 = 10240
E = 160000
E2 = 163840
DIN = 256
DHID = 512
DOUT = 256
HALF = 128
NT = 16
_f32 = jnp.float32

CH = 64
NB = 8
NCHUNK = E2 // (NT * CH)
NBLK = NCHUNK // NB
RPT = N2 // NT


def _sc_agg_body(table, srcs, dsts, dep, agg_out, accum, sidx, didx,
                 gb0, gb1, gb2, gb3, gs0, gs1, gs2, gs3,
                 ss0, ss1, ss2, ss3, isem):
    del dep
    c = lax.axis_index("c")
    s = lax.axis_index("s")
    base = s * RPT
    GB = (gb0, gb1, gb2, gb3)
    GS = (gs0, gs1, gs2, gs3)
    SS = (ss0, ss1, ss2, ss3)

    def _zrow(i, _):
        for k in range(HALF // 16):
            gb0[i, pl.ds(k * 16, 16)] = jnp.zeros((16,), _f32)
        return 0
    lax.fori_loop(0, CH, _zrow, 0)
    for q in range(RPT // CH):
        pltpu.sync_copy(gb0, accum.at[pl.ds(base + q * CH, CH)])

    plsc.subcore_barrier()

    def _idx_start(b, par):
        pltpu.async_copy(srcs.at[s, pl.ds(b * NB, NB)], sidx.at[par], isem)
        pltpu.async_copy(dsts.at[s, pl.ds(b * NB, NB)], didx.at[par], isem)

    def _idx_wait(b, par):
        pltpu.make_async_copy(srcs.at[s, pl.ds(b * NB, NB)], sidx.at[par],
                              isem).wait()
        pltpu.make_async_copy(dsts.at[s, pl.ds(b * NB, NB)], didx.at[par],
                              isem).wait()

    _idx_start(0, 0)

    def _blk(b, _):
        par = lax.rem(b, 2)
        _idx_wait(b, par)

        @pl.when(b + 1 < NBLK)
        def _():
            _idx_start(b + 1, 1 - par)

        si = sidx.at[par]
        di = didx.at[par]

        def _fix(j, _):
            for k in range(CH // 16):
                sl = si[j, pl.ds(k * 16, 16)]
                si[j, pl.ds(k * 16, 16)] = sl * 2 + c
            return 0
        lax.fori_loop(0, NB, _fix, 0)

        def _swait(l):
            pltpu.make_async_copy(GB[l % 4], accum.at[di.at[0]],
                                  SS[l % 4]).wait()

        for l in (0, 1):
            @pl.when(b > 0)
            def _():
                _swait(l)
            pltpu.async_copy(table.at[si.at[l]], GB[l], GS[l])

        for l in range(NB):
            X = l % 4
            pltpu.make_async_copy(table.at[si.at[l]], GB[X], GS[X]).wait()
            pltpu.async_copy(GB[X], accum.at[di.at[l]], SS[X], add=True)
            if l < NB - 2:
                Y = (l + 2) % 4

                @pl.when((b > 0) | (l >= 2))
                def _():
                    _swait(l + 2)
                pltpu.async_copy(table.at[si.at[l + 2]], GB[Y], GS[Y])
        return 0
    lax.fori_loop(0, NBLK, _blk, 0)

    for l in range(4):
        pltpu.make_async_copy(GB[l], accum.at[didx.at[1, 0]], SS[l]).wait()

    plsc.subcore_barrier()

    pltpu.sync_copy(accum.at[pl.ds(base, RPT)],
                    agg_out.at[pl.ds(base, RPT), c])


def _sc_deg_body(dsts, deg_out, deg2d, didx, ones_v, dsem):
    c = lax.axis_index("c")
    s = lax.axis_index("s")
    base = s * RPT

    def _zrow(i, _):
        for k in range(HALF // 16):
            ones_v[i, pl.ds(k * 16, 16)] = jnp.zeros((16,), _f32)
        return 0
    lax.fori_loop(0, CH, _zrow, 0)
    for q in range(RPT // CH):
        pltpu.sync_copy(ones_v, deg2d.at[pl.ds(base + q * CH, CH)])

    def _orow(i, _):
        for k in range(HALF // 16):
            ones_v[i, pl.ds(k * 16, 16)] = jnp.full((16,), 1.0, _f32)
        return 0
    lax.fori_loop(0, CH, _orow, 0)

    plsc.subcore_barrier()

    def _blk(b, _):
        blk = c * (NBLK // 2) + b
        pltpu.sync_copy(dsts.at[s, pl.ds(blk * NB, NB)], didx)
        for j in range(NB):
            pltpu.async_copy(ones_v, deg2d.at[didx.at[j]], dsem, add=True)
        for j in range(NB):
            pltpu.make_async_copy(ones_v, deg2d.at[didx.at[j]], dsem).wait()
        return 0
    lax.fori_loop(0, NBLK // 2, _blk, 0)

    plsc.subcore_barrier()

    pltpu.sync_copy(deg2d.at[pl.ds(base, RPT)],
                    deg_out.at[pl.ds(base, RPT), c])


_SC_MESH = plsc.VectorSubcoreMesh(core_axis_name="c", subcore_axis_name="s")

_sc_agg = pl.kernel(
    _sc_agg_body,
    out_type=[jax.ShapeDtypeStruct((N2, 2, HALF), _f32)],
    mesh=_SC_MESH,
    scratch_types=[
        pltpu.VMEM_SHARED((N2, HALF), _f32),
        pltpu.VMEM((2, NB, CH), jnp.int32),
        pltpu.VMEM((2, NB, CH), jnp.int32),
        pltpu.VMEM((CH, HALF), _f32),
        pltpu.VMEM((CH, HALF), _f32),
        pltpu.VMEM((CH, HALF), _f32),
        pltpu.VMEM((CH, HALF), _f32),
        pltpu.SemaphoreType.DMA,
        pltpu.SemaphoreType.DMA,
        pltpu.SemaphoreType.DMA,
        pltpu.SemaphoreType.DMA,
        pltpu.SemaphoreType.DMA,
        pltpu.SemaphoreType.DMA,
        pltpu.SemaphoreType.DMA,
        pltpu.SemaphoreType.DMA,
        pltpu.SemaphoreType.DMA,
    ],
)

_sc_deg = pl.kernel(
    _sc_deg_body,
    out_type=[jax.ShapeDtypeStruct((N2, 2, HALF), _f32)],
    mesh=_SC_MESH,
    scratch_types=[
        pltpu.VMEM_SHARED((N2, HALF), _f32),
        pltpu.VMEM((NB, CH), jnp.int32),
        pltpu.VMEM((CH, HALF), _f32),
        pltpu.SemaphoreType.DMA,
    ],
)


BR = 400
_bf16 = jnp.bfloat16


def _tcinv_body(deg, inv):
    inv[...] = 1.0 / jnp.maximum(deg[:, 0, 0:1] + deg[:, 1, 0:1], 1.0)


def _tcinv(deg):
    return pl.pallas_call(
        _tcinv_body,
        grid=(N // BR,),
        in_specs=[pl.BlockSpec((BR, 2, HALF), lambda i: (i, 0, 0))],
        out_specs=pl.BlockSpec((BR, 1), lambda i: (i, 0)),
        out_shape=jax.ShapeDtypeStruct((N, 1), _f32),
    )(deg)


def _tc1_body(a, inv, x, wl, wr, b, w2l, h_out, p_out):
    iv = inv[...]
    acc = jnp.dot((a[:, 0, :] * iv).astype(_bf16), wl[:HALF],
                  preferred_element_type=_f32)
    acc += jnp.dot((a[:, 1, :] * iv).astype(_bf16), wl[HALF:],
                   preferred_element_type=_f32)
    acc += jnp.dot(x[...].astype(_bf16), wr[...], preferred_element_type=_f32)
    hb = jnp.maximum(acc + b[...], 0.0).astype(_bf16)
    h_out[...] = hb
    pp = jnp.dot(hb, w2l[...], preferred_element_type=_f32)
    p_out[:, 0, :] = pp[:, :HALF]
    p_out[:, 1, :] = pp[:, HALF:]


def _tc1(agg, inv, x, w1l, w1r, b1, w2l):
    return pl.pallas_call(
        _tc1_body,
        grid=(N // BR,),
        in_specs=[
            pl.BlockSpec((BR, 2, HALF), lambda i: (i, 0, 0)),
            pl.BlockSpec((BR, 1), lambda i: (i, 0)),
            pl.BlockSpec((BR, DIN), lambda i: (i, 0)),
            pl.BlockSpec((DIN, DHID), lambda i: (0, 0)),
            pl.BlockSpec((DIN, DHID), lambda i: (0, 0)),
            pl.BlockSpec((1, DHID), lambda i: (0, 0)),
            pl.BlockSpec((DHID, DOUT), lambda i: (0, 0)),
        ],
        out_specs=[
            pl.BlockSpec((BR, DHID), lambda i: (i, 0)),
            pl.BlockSpec((BR, 2, HALF), lambda i: (i, 0, 0)),
        ],
        out_shape=[
            jax.ShapeDtypeStruct((N, DHID), _bf16),
            jax.ShapeDtypeStruct((N, 2, HALF), _f32),
        ],
    )(agg, inv, x, w1l, w1r, b1, w2l)


def _tc2r_body(h, wr, b, r_out):
    rr = jnp.dot(h[...], wr[...], preferred_element_type=_f32) + b[...]
    r_out[:, 0, :] = rr[:, :HALF].astype(_bf16)
    r_out[:, 1, :] = rr[:, HALF:].astype(_bf16)


def _tc2r(h, w2r, b2):
    return pl.pallas_call(
        _tc2r_body,
        grid=(N // BR,),
        in_specs=[
            pl.BlockSpec((BR, DHID), lambda i: (i, 0)),
            pl.BlockSpec((DHID, DOUT), lambda i: (0, 0)),
            pl.BlockSpec((1, DOUT), lambda i: (0, 0)),
        ],
        out_specs=pl.BlockSpec((BR, 2, HALF), lambda i: (i, 0, 0)),
        out_shape=jax.ShapeDtypeStruct((N, 2, HALF), _bf16),
    )(h, w2r, b2)


def _tc3_body(a, inv, r, z):
    iv = inv[...]
    rr = r[...].astype(_f32)
    z[...] = jnp.concatenate(
        [a[:, 0, :] * iv + rr[:, 0, :], a[:, 1, :] * iv + rr[:, 1, :]],
        axis=1)


def _tc3(agg2, inv, r3):
    return pl.pallas_call(
        _tc3_body,
        grid=(N // BR,),
        in_specs=[
            pl.BlockSpec((BR, 2, HALF), lambda i: (i, 0, 0)),
            pl.BlockSpec((BR, 1), lambda i: (i, 0)),
            pl.BlockSpec((BR, 2, HALF), lambda i: (i, 0, 0)),
        ],
        out_specs=pl.BlockSpec((BR, DOUT), lambda i: (i, 0)),
        out_shape=jax.ShapeDtypeStruct((N, DOUT), _f32),
    )(agg2, inv, r3)


def kernel(x, edge_index, W1_l, W1_r, b1, W2_l, W2_r, b2):
    src = edge_index[0].astype(jnp.int32)
    dst = edge_index[1].astype(jnp.int32)
    pad = jnp.arange(E2 - E, dtype=jnp.int32) % 16
    srcs = jnp.concatenate([src, pad]).reshape(NT, NCHUNK, CH)
    dsts = jnp.concatenate([dst, pad + N]).reshape(NT, NCHUNK, CH)

    xflat = x.reshape(2 * N, HALF)

    (deg,) = _sc_deg(dsts)
    inv = _tcinv(deg)
    (agg1,) = _sc_agg(xflat, srcs, dsts, deg)
    h, p3 = _tc1(agg1, inv, x, W1_l.astype(_bf16), W1_r.astype(_bf16),
                 b1.reshape(1, DHID), W2_l.astype(_bf16))
    r3 = _tc2r(h, W2_r.astype(_bf16), b2.reshape(1, DOUT))
    (agg2,) = _sc_agg(p3.reshape(2 * N, HALF), srcs, dsts, deg)
    return _tc3(agg2, inv, r3)


# speedup vs baseline: 1.2315x; 1.0092x over previous
import jax
import jax.numpy as jnp
from jax import lax
from jax.experimental import pallas as pl
from jax.experimental.pallas import tpu as pltpu
from jax.experimental.pallas import tpu_sc as plsc

N = 10000
N2 = 10240
E = 160000
E2 = 163840
DIN = 256
DHID = 512
DOUT = 256
HALF = 128
NT = 16
_f32 = jnp.float32

CH = 80
NB = 8
NCHUNK = E2 // (NT * CH)
NBLK = NCHUNK // NB
RPT = N2 // NT


def _sc_agg_body(table, srcs, dsts, dep, agg_out, accum, sidx, didx,
                 gb0, gb1, gb2, gb3, gs0, gs1, gs2, gs3,
                 ss0, ss1, ss2, ss3, isem):
    del dep
    c = lax.axis_index("c")
    s = lax.axis_index("s")
    base = s * RPT
    GB = (gb0, gb1, gb2, gb3)
    GS = (gs0, gs1, gs2, gs3)
    SS = (ss0, ss1, ss2, ss3)

    def _zrow(i, _):
        for k in range(HALF // 16):
            gb0[i, pl.ds(k * 16, 16)] = jnp.zeros((16,), _f32)
        return 0
    lax.fori_loop(0, CH, _zrow, 0)
    for q in range(RPT // CH):
        pltpu.sync_copy(gb0, accum.at[pl.ds(base + q * CH, CH)])

    plsc.subcore_barrier()

    def _idx_start(b, par):
        pltpu.async_copy(srcs.at[s, pl.ds(b * NB, NB)], sidx.at[par], isem)
        pltpu.async_copy(dsts.at[s, pl.ds(b * NB, NB)], didx.at[par], isem)

    def _idx_wait(b, par):
        pltpu.make_async_copy(srcs.at[s, pl.ds(b * NB, NB)], sidx.at[par],
                              isem).wait()
        pltpu.make_async_copy(dsts.at[s, pl.ds(b * NB, NB)], didx.at[par],
                              isem).wait()

    _idx_start(0, 0)

    def _blk(b, _):
        par = lax.rem(b, 2)
        _idx_wait(b, par)

        @pl.when(b + 1 < NBLK)
        def _():
            _idx_start(b + 1, 1 - par)

        si = sidx.at[par]
        di = didx.at[par]

        def _fix(j, _):
            for k in range(CH // 16):
                sl = si[j, pl.ds(k * 16, 16)]
                si[j, pl.ds(k * 16, 16)] = sl * 2 + c
            return 0
        lax.fori_loop(0, NB, _fix, 0)

        def _swait(l):
            pltpu.make_async_copy(GB[l % 4], accum.at[di.at[0]],
                                  SS[l % 4]).wait()

        for l in (0, 1):
            @pl.when(b > 0)
            def _():
                _swait(l)
            pltpu.async_copy(table.at[si.at[l]], GB[l], GS[l])

        for l in range(NB):
            X = l % 4
            pltpu.make_async_copy(table.at[si.at[l]], GB[X], GS[X]).wait()
            pltpu.async_copy(GB[X], accum.at[di.at[l]], SS[X], add=True)
            if l < NB - 2:
                Y = (l + 2) % 4

                @pl.when((b > 0) | (l >= 2))
                def _():
                    _swait(l + 2)
                pltpu.async_copy(table.at[si.at[l + 2]], GB[Y], GS[Y])
        return 0
    lax.fori_loop(0, NBLK, _blk, 0)

    for l in range(4):
        pltpu.make_async_copy(GB[l], accum.at[didx.at[1, 0]], SS[l]).wait()

    plsc.subcore_barrier()

    pltpu.sync_copy(accum.at[pl.ds(base, RPT)],
                    agg_out.at[pl.ds(base, RPT), c])


def _sc_deg_body(dsts, deg_out, deg2d, didx, ones_v, dsem):
    c = lax.axis_index("c")
    s = lax.axis_index("s")
    base = s * RPT

    def _zrow(i, _):
        for k in range(HALF // 16):
            ones_v[i, pl.ds(k * 16, 16)] = jnp.zeros((16,), _f32)
        return 0
    lax.fori_loop(0, CH, _zrow, 0)
    for q in range(RPT // CH):
        pltpu.sync_copy(ones_v, deg2d.at[pl.ds(base + q * CH, CH)])

    def _orow(i, _):
        for k in range(HALF // 16):
            ones_v[i, pl.ds(k * 16, 16)] = jnp.full((16,), 1.0, _f32)
        return 0
    lax.fori_loop(0, CH, _orow, 0)

    plsc.subcore_barrier()

    def _blk(b, _):
        blk = c * (NBLK // 2) + b
        pltpu.sync_copy(dsts.at[s, pl.ds(blk * NB, NB)], didx)
        for j in range(NB):
            pltpu.async_copy(ones_v, deg2d.at[didx.at[j]], dsem, add=True)
        for j in range(NB):
            pltpu.make_async_copy(ones_v, deg2d.at[didx.at[j]], dsem).wait()
        return 0
    lax.fori_loop(0, NBLK // 2, _blk, 0)

    plsc.subcore_barrier()

    pltpu.sync_copy(deg2d.at[pl.ds(base, RPT)],
                    deg_out.at[pl.ds(base, RPT), c])


_SC_MESH = plsc.VectorSubcoreMesh(core_axis_name="c", subcore_axis_name="s")

_sc_agg = pl.kernel(
    _sc_agg_body,
    out_type=[jax.ShapeDtypeStruct((N2, 2, HALF), _f32)],
    mesh=_SC_MESH,
    scratch_types=[
        pltpu.VMEM_SHARED((N2, HALF), _f32),
        pltpu.VMEM((2, NB, CH), jnp.int32),
        pltpu.VMEM((2, NB, CH), jnp.int32),
        pltpu.VMEM((CH, HALF), _f32),
        pltpu.VMEM((CH, HALF), _f32),
        pltpu.VMEM((CH, HALF), _f32),
        pltpu.VMEM((CH, HALF), _f32),
        pltpu.SemaphoreType.DMA,
        pltpu.SemaphoreType.DMA,
        pltpu.SemaphoreType.DMA,
        pltpu.SemaphoreType.DMA,
        pltpu.SemaphoreType.DMA,
        pltpu.SemaphoreType.DMA,
        pltpu.SemaphoreType.DMA,
        pltpu.SemaphoreType.DMA,
        pltpu.SemaphoreType.DMA,
    ],
)

_sc_deg = pl.kernel(
    _sc_deg_body,
    out_type=[jax.ShapeDtypeStruct((N2, 2, HALF), _f32)],
    mesh=_SC_MESH,
    scratch_types=[
        pltpu.VMEM_SHARED((N2, HALF), _f32),
        pltpu.VMEM((NB, CH), jnp.int32),
        pltpu.VMEM((CH, HALF), _f32),
        pltpu.SemaphoreType.DMA,
    ],
)


BR = 400
_bf16 = jnp.bfloat16


def _tcinv_body(deg, inv):
    inv[...] = 1.0 / jnp.maximum(deg[:, 0, 0:1] + deg[:, 1, 0:1], 1.0)


def _tcinv(deg):
    return pl.pallas_call(
        _tcinv_body,
        grid=(N // BR,),
        in_specs=[pl.BlockSpec((BR, 2, HALF), lambda i: (i, 0, 0))],
        out_specs=pl.BlockSpec((BR, 1), lambda i: (i, 0)),
        out_shape=jax.ShapeDtypeStruct((N, 1), _f32),
    )(deg)


def _tc1_body(a, inv, x, wl, wr, b, w2l, h_out, p_out):
    iv = inv[...]
    acc = jnp.dot((a[:, 0, :] * iv).astype(_bf16), wl[:HALF],
                  preferred_element_type=_f32)
    acc += jnp.dot((a[:, 1, :] * iv).astype(_bf16), wl[HALF:],
                   preferred_element_type=_f32)
    acc += jnp.dot(x[...], wr[...], preferred_element_type=_f32)
    hb = jnp.maximum(acc + b[...], 0.0).astype(_bf16)
    h_out[...] = hb
    pp = jnp.dot(hb, w2l[...], preferred_element_type=_f32)
    p_out[:, 0, :] = pp[:, :HALF]
    p_out[:, 1, :] = pp[:, HALF:]


def _tc1(agg, inv, x, w1l, w1r, b1, w2l):
    return pl.pallas_call(
        _tc1_body,
        grid=(N // BR,),
        in_specs=[
            pl.BlockSpec((BR, 2, HALF), lambda i: (i, 0, 0)),
            pl.BlockSpec((BR, 1), lambda i: (i, 0)),
            pl.BlockSpec((BR, DIN), lambda i: (i, 0)),
            pl.BlockSpec((DIN, DHID), lambda i: (0, 0)),
            pl.BlockSpec((DIN, DHID), lambda i: (0, 0)),
            pl.BlockSpec((1, DHID), lambda i: (0, 0)),
            pl.BlockSpec((DHID, DOUT), lambda i: (0, 0)),
        ],
        out_specs=[
            pl.BlockSpec((BR, DHID), lambda i: (i, 0)),
            pl.BlockSpec((BR, 2, HALF), lambda i: (i, 0, 0)),
        ],
        out_shape=[
            jax.ShapeDtypeStruct((N, DHID), _bf16),
            jax.ShapeDtypeStruct((N, 2, HALF), _f32),
        ],
    )(agg, inv, x, w1l, w1r, b1, w2l)


def _tc2r_body(h, wr, b, r_out):
    rr = jnp.dot(h[...], wr[...], preferred_element_type=_f32) + b[...]
    r_out[:, 0, :] = rr[:, :HALF].astype(_bf16)
    r_out[:, 1, :] = rr[:, HALF:].astype(_bf16)


def _tc2r(h, w2r, b2):
    return pl.pallas_call(
        _tc2r_body,
        grid=(N // BR,),
        in_specs=[
            pl.BlockSpec((BR, DHID), lambda i: (i, 0)),
            pl.BlockSpec((DHID, DOUT), lambda i: (0, 0)),
            pl.BlockSpec((1, DOUT), lambda i: (0, 0)),
        ],
        out_specs=pl.BlockSpec((BR, 2, HALF), lambda i: (i, 0, 0)),
        out_shape=jax.ShapeDtypeStruct((N, 2, HALF), _bf16),
    )(h, w2r, b2)


def _tc3_body(a, inv, r, z):
    iv = inv[...]
    rr = r[...].astype(_f32)
    z[...] = jnp.concatenate(
        [a[:, 0, :] * iv + rr[:, 0, :], a[:, 1, :] * iv + rr[:, 1, :]],
        axis=1)


def _tc3(agg2, inv, r3):
    return pl.pallas_call(
        _tc3_body,
        grid=(N // BR,),
        in_specs=[
            pl.BlockSpec((BR, 2, HALF), lambda i: (i, 0, 0)),
            pl.BlockSpec((BR, 1), lambda i: (i, 0)),
            pl.BlockSpec((BR, 2, HALF), lambda i: (i, 0, 0)),
        ],
        out_specs=pl.BlockSpec((BR, DOUT), lambda i: (i, 0)),
        out_shape=jax.ShapeDtypeStruct((N, DOUT), _f32),
    )(agg2, inv, r3)


def kernel(x, edge_index, W1_l, W1_r, b1, W2_l, W2_r, b2):
    src = edge_index[0].astype(jnp.int32)
    dst = edge_index[1].astype(jnp.int32)
    pad = jnp.arange(E2 - E, dtype=jnp.int32) % 16
    srcs = jnp.concatenate([src, pad]).reshape(NT, NCHUNK, CH)
    dsts = jnp.concatenate([dst, pad + N]).reshape(NT, NCHUNK, CH)

    xflat = x.reshape(2 * N, HALF)

    (deg,) = _sc_deg(dsts)
    inv = _tcinv(deg)
    (agg1,) = _sc_agg(xflat, srcs, dsts, deg)
    h, p3 = _tc1(agg1, inv, x.astype(_bf16), W1_l.astype(_bf16),
                 W1_r.astype(_bf16), b1.reshape(1, DHID), W2_l.astype(_bf16))
    r3 = _tc2r(h, W2_r.astype(_bf16), b2.reshape(1, DOUT))
    (agg2,) = _sc_agg(p3.reshape(2 * N, HALF), srcs, dsts, deg)
    return _tc3(agg2, inv, r3)
